# baseline (speedup 1.0000x reference)
import functools

import jax
import jax.numpy as jnp
from jax import lax
from jax.experimental import pallas as pl
from jax.experimental.pallas import tpu as pltpu
from jax.experimental.pallas import tpu_sc as plsc

VOCAB_N = 1_000_000
D = 64
BATCH_N = 16384
HIST_N = 200
B_TOTAL = BATCH_N * HIST_N

NW = 32
LANES = 16
BBLK = 128
NBT = BATCH_N // BBLK
BT_W = NBT // NW
NUNIT = BT_W * HIST_N
NRING = 4

_MEAN_BLK = 8000
_MEAN_GRID = VOCAB_N // _MEAN_BLK


def _prep_body(t_ref, c_ref):
    i = pl.program_id(0)

    @pl.when(i == 0)
    def _():
        c_ref[...] = jnp.zeros_like(c_ref)

    s = jnp.sum(t_ref[...], axis=0, keepdims=True)
    c_ref[...] += jnp.broadcast_to(jnp.concatenate([s, s], axis=1), (8, 2 * D))

    @pl.when(i == _MEAN_GRID - 1)
    def _():
        c_ref[...] = c_ref[...] * (1.0 / VOCAB_N)


def _tc_prep(table):
    return pl.pallas_call(
        _prep_body,
        grid=(_MEAN_GRID,),
        in_specs=[pl.BlockSpec((_MEAN_BLK, D), lambda i: (i, 0))],
        out_specs=pl.BlockSpec((8, 2 * D), lambda i: (0, 0)),
        out_shape=jax.ShapeDtypeStruct((8, 2 * D), jnp.float32),
    )(table)


_mesh = plsc.VectorSubcoreMesh(core_axis_name="c", subcore_axis_name="s")


@functools.partial(
    pl.kernel,
    mesh=_mesh,
    compiler_params=pltpu.CompilerParams(
        use_tc_tiling_on_sc=True, needs_layout_passes=False),
    out_type=jax.ShapeDtypeStruct((HIST_N, 8, NBT, 8, BBLK), jnp.float32),
    scratch_types=[
        pltpu.VMEM((NRING, 1, BBLK), jnp.int32),
        pltpu.VMEM((NRING, 1, BBLK), jnp.int32),
        pltpu.VMEM((NRING, BBLK, 2 * D), jnp.float32),
        pltpu.VMEM((NRING, 8, 8, BBLK), jnp.float32),
        pltpu.VMEM((D, BBLK), jnp.float32),
        pltpu.SemaphoreType.DMA,
        pltpu.SemaphoreType.DMA,
        pltpu.SemaphoreType.DMA,
    ],
)
def _sc_gather_sub(xt_hbm, table_hbm, csplat_hbm, out_hbm,
                   idx_v, q_v, rows_v, stage_v, csplat_v,
                   sem_i, sem_g, sem_s):
    wid = lax.axis_index("s") * 2 + lax.axis_index("c")
    bt0 = wid * BT_W

    pltpu.sync_copy(csplat_hbm, csplat_v)
    iotas = [lax.iota(jnp.int32, LANES) + LANES * g for g in range(8)]

    def idx_src(h, bt):
        return xt_hbm.at[h, pl.ds(bt * BBLK, BBLK)]

    def fire_idx(slot, h, bt):
        pltpu.async_copy(idx_src(h, bt), idx_v.at[slot, 0], sem_i)

    def wait_idx(slot):
        pltpu.make_async_copy(idx_src(0, 0), idx_v.at[slot, 0], sem_i).wait()

    def compute_q(slot):
        for g in range(8):
            sl = pl.ds(LANES * g, LANES)
            q_v[slot, 0, sl] = lax.shift_right_logical(idx_v[slot, 0, sl], 1)

    def fire_gather(slot):
        pltpu.async_copy(table_hbm.at[q_v.at[slot, 0]], rows_v.at[slot], sem_g)

    def wait_gather(slot):
        pltpu.make_async_copy(
            table_hbm.at[q_v.at[slot, 0]], rows_v.at[slot], sem_g).wait()

    def fire_scatter(slot, h, bt):
        for dt in range(8):
            pltpu.async_copy(
                stage_v.at[slot, dt], out_hbm.at[h, dt, bt], sem_s)

    def wait_scatter(slot):
        for dt in range(8):
            pltpu.make_async_copy(
                stage_v.at[slot, dt], out_hbm.at[0, dt, 0], sem_s).wait()

    def process(slot):
        parvecs = [(idx_v[slot, 0, pl.ds(LANES * g, LANES)] & 1) * D
                   for g in range(8)]

        def drow(d, dvec):
            dt = lax.shift_right_logical(d, 3)
            di = lax.rem(d, 8)
            csp = csplat_v[d, pl.ds(0, LANES)]
            for g in range(8):
                v = plsc.load_gather(rows_v.at[slot], [iotas[g], parvecs[g] + dvec])
                stage_v[slot, dt, di, pl.ds(LANES * g, LANES)] = v - csp
            return dvec + 1

        lax.fori_loop(0, D, drow, jnp.zeros((LANES,), jnp.int32))

    for v in range(2):
        fire_idx(v, v, bt0)
        wait_idx(v)
    for v in range(2):
        compute_q(v)
        fire_gather(v)
    fire_idx(2, 2, bt0)

    def body(u, carry):
        h, bt, h3, bt3 = carry
        slot = lax.rem(u, NRING)

        @pl.when(u + 3 < NUNIT)
        def _():
            fire_idx(lax.rem(u + 3, NRING), h3, bt3)

        @pl.when(u + 2 < NUNIT)
        def _():
            s2 = lax.rem(u + 2, NRING)
            wait_idx(s2)
            compute_q(s2)
            fire_gather(s2)

        wait_gather(slot)
        process(slot)
        fire_scatter(slot, h, bt)

        @pl.when(u >= 2)
        def _():
            wait_scatter(lax.rem(u + 2, NRING))

        h_n = h + 1
        wrap = h_n == HIST_N
        h_n = lax.select(wrap, 0, h_n)
        bt_n = lax.select(wrap, bt + 1, bt)
        h3_n = h3 + 1
        wrap3 = h3_n == HIST_N
        h3_n = lax.select(wrap3, 0, h3_n)
        bt3_n = lax.select(wrap3, bt3 + 1, bt3)
        return (h_n, bt_n, h3_n, bt3_n)

    lax.fori_loop(0, NUNIT, body,
                  (0, bt0, 3 % HIST_N, bt0 + 3 // HIST_N))
    wait_scatter(0)
    wait_scatter(1)


def kernel(x, table):
    center = _tc_prep(table)
    table2 = table.reshape(VOCAB_N // 2, 2 * D)
    c64 = center[0, :D]
    csplat = jnp.broadcast_to(c64[:, None], (D, BBLK))
    xt = jnp.transpose(x.astype(jnp.int32))
    out5 = _sc_gather_sub(xt, table2, csplat)
    return out5.transpose(2, 4, 0, 1, 3).reshape(BATCH_N, HIST_N, D)


# speedup vs baseline: 1.7464x; 1.7464x over previous
import functools

import jax
import jax.numpy as jnp
from jax import lax
from jax.experimental import pallas as pl
from jax.experimental.pallas import tpu as pltpu
from jax.experimental.pallas import tpu_sc as plsc

VOCAB_N = 1_000_000
D = 64
BATCH_N = 16384
HIST_N = 200
B_TOTAL = BATCH_N * HIST_N

NW = 32
PER_W = B_TOTAL // NW
SUB = 128
NSUB = PER_W // SUB
LANES = 16
NCREG = D // LANES
NRING = 3

_MEAN_BLK = 8000
_MEAN_GRID = VOCAB_N // _MEAN_BLK


def _mean_body(t_ref, c_ref):
    i = pl.program_id(0)

    @pl.when(i == 0)
    def _():
        c_ref[...] = jnp.zeros_like(c_ref)

    s = jnp.sum(t_ref[...], axis=0, keepdims=True)
    c_ref[...] += jnp.broadcast_to(jnp.concatenate([s, s], axis=1), (8, 2 * D))

    @pl.when(i == _MEAN_GRID - 1)
    def _():
        c_ref[...] = c_ref[...] * (1.0 / VOCAB_N)


def _tc_mean(table):
    return pl.pallas_call(
        _mean_body,
        grid=(_MEAN_GRID,),
        in_specs=[pl.BlockSpec((_MEAN_BLK, D), lambda i: (i, 0))],
        out_specs=pl.BlockSpec((8, 2 * D), lambda i: (0, 0)),
        out_shape=jax.ShapeDtypeStruct((8, 2 * D), jnp.float32),
    )(table)


_mesh = plsc.VectorSubcoreMesh(core_axis_name="c", subcore_axis_name="s")


@functools.partial(
    pl.kernel,
    mesh=_mesh,
    compiler_params=pltpu.CompilerParams(
        use_tc_tiling_on_sc=False, needs_layout_passes=False),
    out_type=jax.ShapeDtypeStruct((B_TOTAL, 2 * D), jnp.float32),
    scratch_types=[
        pltpu.VMEM((NRING, 1, SUB), jnp.int32),
        pltpu.VMEM((NRING, 1, SUB), jnp.int32),
        pltpu.VMEM((NRING, 1, SUB), jnp.int32),
        pltpu.VMEM((NRING, SUB, 2 * D), jnp.float32),
        pltpu.VMEM((NRING, SUB, D), jnp.float32),
        pltpu.VMEM((8, 2 * D), jnp.float32),
        pltpu.SemaphoreType.DMA,
        pltpu.SemaphoreType.DMA,
        pltpu.SemaphoreType.DMA,
    ],
)
def _sc_gather_sub(x_hbm, table_hbm, center_hbm, out_hbm,
                   idx_v, q_v, par_v, rows_v, stage_v, center_v,
                   sem_i, sem_g, sem_s):
    wid = lax.axis_index("s") * 2 + lax.axis_index("c")
    xbase = wid * NSUB
    obase = wid * PER_W

    pltpu.sync_copy(center_hbm, center_v)
    cregs = [center_v[0, pl.ds(LANES * c, LANES)] for c in range(NCREG)]

    def fire_idx(slot, s):
        pltpu.async_copy(x_hbm.at[pl.ds(xbase + s, 1)], idx_v.at[slot], sem_i)

    def wait_idx(slot):
        pltpu.make_async_copy(
            x_hbm.at[pl.ds(0, 1)], idx_v.at[slot], sem_i).wait()

    def compute_qpar(slot):
        for g in range(SUB // LANES):
            sl = pl.ds(LANES * g, LANES)
            v = idx_v[slot, 0, sl]
            q_v[slot, 0, sl] = lax.shift_right_logical(v, 1)
            par_v[slot, 0, sl] = (v & 1) * D

    def fire_gather(slot):
        pltpu.async_copy(table_hbm.at[q_v.at[slot, 0]], rows_v.at[slot], sem_g)

    def wait_gather(slot):
        pltpu.make_async_copy(
            table_hbm.at[q_v.at[slot, 0]], rows_v.at[slot], sem_g).wait()

    def fire_scatter(slot, s):
        pltpu.async_copy(
            stage_v.at[slot],
            out_hbm.at[pl.ds(obase + s * SUB, SUB), pl.ds(0, D)],
            sem_s)

    def wait_scatter(slot):
        pltpu.make_async_copy(
            stage_v.at[slot],
            out_hbm.at[pl.ds(0, SUB), pl.ds(0, D)],
            sem_s).wait()

    def process(slot):
        def blk(bb, carry):
            pv = par_v[slot, 0, pl.ds(LANES * bb, LANES)]
            for i in range(LANES):
                j = LANES * bb + i
                off = pv[i]
                for c in range(NCREG):
                    stage_v[slot, j, pl.ds(LANES * c, LANES)] = (
                        rows_v[slot, j, pl.ds(off + LANES * c, LANES)]
                        - cregs[c])
            return carry

        lax.fori_loop(0, SUB // LANES, blk, 0)

    for v in range(2):
        fire_idx(v, v)
        wait_idx(v)
        compute_qpar(v)
        fire_gather(v)
    fire_idx(2, 2)

    def body(u, carry):
        slot = lax.rem(u, NRING)

        @pl.when(u + 2 < NSUB)
        def _():
            s2 = lax.rem(u + 2, NRING)
            wait_idx(s2)
            compute_qpar(s2)
            fire_gather(s2)

        @pl.when(u + 3 < NSUB)
        def _():
            fire_idx(slot, u + 3)

        wait_gather(slot)

        @pl.when(u >= 2)
        def _():
            wait_scatter(lax.rem(u + 1, NRING))

        process(slot)
        fire_scatter(slot, u)
        return carry

    lax.fori_loop(0, NSUB, body, 0)
    wait_scatter(0)
    wait_scatter(1)


def kernel(x, table):
    center = _tc_mean(table)
    x2 = x.reshape(-1).astype(jnp.int32).reshape(B_TOTAL // SUB, SUB)
    table2 = table.reshape(VOCAB_N // 2, 2 * D)
    out2d = _sc_gather_sub(x2, table2, center)
    return out2d.reshape(BATCH_N, HIST_N, 2 * D)[:, :, :D]
